# Initial kernel scaffold; baseline (speedup 1.0000x reference)
import jax
import jax.numpy as jnp
from jax.experimental import pallas as pl


def kernel(xyz, points, W1, b1, g1, be1, W2, b2, g2, be2, W3, b3, g3, be3):
    raise NotImplementedError("write your pallas kernel here")


# speedup vs baseline: 18.5696x; 18.5696x over previous
import functools

import jax
import jax.numpy as jnp
from jax import lax
from jax.experimental import pallas as pl
from jax.experimental.pallas import tpu as pltpu
from jax.experimental.pallas import tpu_sc as plsc

_NPOINT = 1024
_NSAMPLE = 32
_RADIUS = 0.2
_INTERPRET = False

_BIGKEY = 0x7FFFFFFF
_IDXMASK = 8191
_KEYCLEAR = -8192


def _fps_body(xyzT_ref, f0_ref, idx_ref, dist_ref):
    _, B, SUB, NL = xyzT_ref.shape
    N = SUB * NL
    x0 = xyzT_ref[0]
    x1 = xyzT_ref[1]
    x2 = xyzT_ref[2]
    s_iota = lax.broadcasted_iota(jnp.int32, (B, SUB, NL), 1)
    l_iota = lax.broadcasted_iota(jnp.int32, (B, SUB, NL), 2)
    nflat = s_iota * NL + l_iota
    bcol = lax.broadcasted_iota(jnp.int32, (1, B), 1)
    dist_ref[...] = jnp.full((B, SUB, NL), 1e10, jnp.float32)

    def _redmax(v):
        return jnp.max(jnp.max(v, axis=2, keepdims=True), axis=1, keepdims=True)

    def _redmin(v):
        return jnp.min(jnp.min(v, axis=2, keepdims=True), axis=1, keepdims=True)

    def step(i, f):
        idx_ref[pl.ds(i, 1), :] = f.reshape(1, B) + bcol * N
        onehot = nflat == f
        c0 = _redmax(jnp.where(onehot, x0, -jnp.inf))
        c1 = _redmax(jnp.where(onehot, x1, -jnp.inf))
        c2 = _redmax(jnp.where(onehot, x2, -jnp.inf))
        d = (x0 - c0) ** 2 + (x1 - c1) ** 2 + (x2 - c2) ** 2
        dist = jnp.minimum(dist_ref[...], d)
        dist_ref[...] = dist
        m = _redmax(dist)
        cand = jnp.where(dist == m, nflat, N)
        return _redmin(cand)

    f0 = f0_ref[0, :].reshape(B, 1, 1)
    lax.fori_loop(0, _NPOINT, step, f0, unroll=False)


def _run_fps(xyzT, f0):
    B = xyzT.shape[1]
    return pl.pallas_call(
        _fps_body,
        out_shape=jax.ShapeDtypeStruct((_NPOINT, B), jnp.int32),
        scratch_shapes=[pltpu.VMEM(xyzT.shape[1:], jnp.float32)],
        interpret=_INTERPRET,
    )(xyzT, f0)


def _select_body(q_ref, xyzT_ref, out_ref, key_ref):
    SB = q_ref.shape[1]
    N = xyzT_ref.shape[2]
    b = pl.program_id(0)
    x0 = xyzT_ref[0, 0:1, :]
    x1 = xyzT_ref[0, 1:2, :]
    x2 = xyzT_ref[0, 2:3, :]
    q0 = q_ref[0, :, 0:1]
    q1 = q_ref[0, :, 1:2]
    q2 = q_ref[0, :, 2:3]
    x2sum = x0 * x0 + x1 * x1 + x2 * x2
    q2sum = q0 * q0 + q1 * q1 + q2 * q2

    def rne16(v):
        bv = lax.bitcast_convert_type(v, jnp.int32)
        r = bv + 0x7FFF + jnp.bitwise_and(lax.shift_right_arithmetic(bv, 16), 1)
        return lax.bitcast_convert_type(jnp.bitwise_and(r, -65536), jnp.float32)

    acc = (rne16(q0) * rne16(x0) + rne16(q1) * rne16(x1)
           + rne16(q2) * rne16(x2))
    d2 = (q2sum + x2sum) - 2.0 * acc
    d = jnp.sqrt(jnp.maximum(d2, 0.0))
    n_iota = lax.broadcasted_iota(jnp.int32, (SB, N), 1)
    bits = lax.bitcast_convert_type(d, jnp.int32)
    key_all = jnp.bitwise_or(jnp.bitwise_and(bits, _KEYCLEAR), n_iota)
    nearest = jnp.bitwise_and(
        jnp.min(key_all, axis=1, keepdims=True), _IDXMASK)
    key = jnp.where(d <= jnp.float32(_RADIUS), key_all, _BIGKEY)
    kmin0 = jnp.min(key, axis=1, keepdims=True)
    key_ref[...] = key
    col = lax.broadcasted_iota(jnp.int32, (SB, _NSAMPLE), 1)
    out0 = jnp.zeros((SB, _NSAMPLE), jnp.int32)

    def step(k, carry):
        out, kmin = carry
        sel = jnp.where(kmin != _BIGKEY,
                        jnp.bitwise_and(kmin, _IDXMASK), nearest)
        out = jnp.where(col == k, sel, out)
        kk = key_ref[...]
        upd = jnp.where(kk == kmin, _BIGKEY, kk)
        key_ref[...] = upd
        return out, jnp.min(upd, axis=1, keepdims=True)

    out, _ = lax.fori_loop(0, _NSAMPLE, step, (out0, kmin0), unroll=False)
    out_ref[0] = out + b * N


def _run_select(q_rows, xyzT2, SB=256):
    B, S, _ = q_rows.shape
    N = xyzT2.shape[2]
    return pl.pallas_call(
        _select_body,
        grid=(B, S // SB),
        in_specs=[
            pl.BlockSpec((1, SB, 16), lambda b, s: (b, s, 0)),
            pl.BlockSpec((1, 3, N), lambda b, s: (b, 0, 0)),
        ],
        out_specs=pl.BlockSpec((1, SB, _NSAMPLE), lambda b, s: (b, s, 0)),
        out_shape=jax.ShapeDtypeStruct((B, S, _NSAMPLE), jnp.int32),
        scratch_shapes=[pltpu.VMEM((SB, N), jnp.int32)],
        interpret=_INTERPRET,
    )(q_rows, xyzT2)


def _gather_rows(table, idx):
    M = idx.shape[0]
    D = table.shape[1]
    NW = 32
    rpw = M // NW
    CH = min(128, rpw)
    nch = rpw // CH
    mesh = plsc.VectorSubcoreMesh(core_axis_name="c", subcore_axis_name="s")
    idx2 = idx.reshape(M // CH, CH)

    @functools.partial(
        pl.kernel,
        out_type=jax.ShapeDtypeStruct((M, D), jnp.float32),
        mesh=mesh,
        scratch_types=[
            pltpu.VMEM((nch, CH), jnp.int32),
            pltpu.VMEM((CH, D), jnp.float32),
            pltpu.SemaphoreType.DMA,
        ],
        compiler_params=pltpu.CompilerParams(use_tc_tiling_on_sc=False),
    )
    def k(table_hbm, idx_hbm, out_hbm, idx_v, rows_v, sem):
        wid = lax.axis_index("s") * 2 + lax.axis_index("c")
        base = wid * rpw
        pltpu.sync_copy(idx_hbm.at[pl.ds(wid * nch, nch)], idx_v)

        def chunk(j, _):
            pltpu.async_copy(table_hbm.at[idx_v.at[j]], rows_v, sem).wait()
            pltpu.sync_copy(rows_v, out_hbm.at[pl.ds(base + j * CH, CH)])
            return 0

        lax.fori_loop(0, nch, chunk, 0, unroll=False)

    return k(table, idx2)


_EPS = 1e-5

def _mxu_dot(a, b):
    return jnp.dot(a.astype(jnp.bfloat16), b.astype(jnp.bfloat16),
                   preferred_element_type=jnp.float32)


def _bn_affine(mean_in, gram_in, wt, bvec, g, be, m_count):
    mu_in = mean_in / m_count
    mu = jnp.dot(mu_in, wt, preferred_element_type=jnp.float32) + bvec
    t = jnp.dot(gram_in / m_count, wt, preferred_element_type=jnp.float32)
    ex2 = (jnp.sum(wt * t, axis=0, keepdims=True)
           + 2.0 * bvec * (mu - bvec) + bvec * bvec)
    var = ex2 - mu * mu
    scale = g / jnp.sqrt(var + _EPS)
    shift = be - mu * scale
    return scale, shift


def _k1_body(g_ref, c_ref, f_ref, sum_ref, gram_ref):
    R = g_ref.shape[0]
    rows = R // _NSAMPLE
    c = c_ref[...]
    cmask = jnp.where(
        lax.broadcasted_iota(jnp.int32, c.shape, 1) < 3, c, 0.0)
    f = (g_ref[...].reshape(rows, _NSAMPLE, 16) - cmask[:, None, :])
    f2 = f.reshape(R, 16)
    f_ref[...] = f2

    @pl.when(pl.program_id(0) == 0)
    def _():
        sum_ref[...] = jnp.zeros_like(sum_ref)
        gram_ref[...] = jnp.zeros_like(gram_ref)

    sum_ref[...] += jnp.sum(f2, axis=0, keepdims=True)
    gram_ref[...] += lax.dot_general(
        f2, f2, (((0,), (0,)), ((), ())),
        preferred_element_type=jnp.float32)


def _stats_body(nlayer, f_ref, w1_ref, b1_ref, g1_ref, be1_ref,
                w2_ref, b2_ref, g2_ref, be2_ref,
                sumf_ref, gramf_ref, s1_ref, g1m_ref,
                sy_ref, gy_ref, m_count=None):
    y = f_ref[...]
    s, t = _bn_affine(sumf_ref[...], gramf_ref[...], w1_ref[...],
                      b1_ref[...], g1_ref[...], be1_ref[...], m_count)
    y = jnp.maximum(
        _mxu_dot(y, w1_ref[...]) * s + t,
        0.0)
    if nlayer == 2:
        s, t = _bn_affine(s1_ref[...], g1m_ref[...], w2_ref[...],
                          b2_ref[...], g2_ref[...], be2_ref[...], m_count)
        y = jnp.maximum(
            _mxu_dot(y, w2_ref[...]) * s
            + t, 0.0)

    @pl.when(pl.program_id(0) == 0)
    def _():
        sy_ref[...] = jnp.zeros_like(sy_ref)
        gy_ref[...] = jnp.zeros_like(gy_ref)

    sy_ref[...] += jnp.sum(y, axis=0, keepdims=True)
    gy_ref[...] += lax.dot_general(
        y, y, (((0,), (0,)), ((), ())), preferred_element_type=jnp.float32)


def _final_body(f_ref, w1_ref, b1_ref, g1_ref, be1_ref,
                w2_ref, b2_ref, g2_ref, be2_ref,
                w3_ref, b3_ref, g3_ref, be3_ref,
                sumf_ref, gramf_ref, s1_ref, g1m_ref, s2_ref, g2m_ref,
                out_ref, m_count=None):
    y = f_ref[...]
    s, t = _bn_affine(sumf_ref[...], gramf_ref[...], w1_ref[...],
                      b1_ref[...], g1_ref[...], be1_ref[...], m_count)
    y = jnp.maximum(
        _mxu_dot(y, w1_ref[...]) * s + t,
        0.0)
    s, t = _bn_affine(s1_ref[...], g1m_ref[...], w2_ref[...],
                      b2_ref[...], g2_ref[...], be2_ref[...], m_count)
    y = jnp.maximum(
        _mxu_dot(y, w2_ref[...]) * s + t,
        0.0)
    s, t = _bn_affine(s2_ref[...], g2m_ref[...], w3_ref[...],
                      b3_ref[...], g3_ref[...], be3_ref[...], m_count)
    y = jnp.maximum(
        _mxu_dot(y, w3_ref[...]) * s + t,
        0.0)
    R = y.shape[0]
    out_ref[...] = jnp.max(
        y.reshape(R // _NSAMPLE, _NSAMPLE, y.shape[1]), axis=1)


def _run_mlp(G, C, w1t, b1, g1, be1, w2t, b2, g2, be2, w3t, b3, g3, be3):
    M = G.shape[0]
    R = 8192
    nblk = M // R
    rows = R // _NSAMPLE
    fM = float(M)

    f, sumf, gramf = pl.pallas_call(
        _k1_body,
        grid=(nblk,),
        in_specs=[
            pl.BlockSpec((R, 16), lambda i: (i, 0)),
            pl.BlockSpec((rows, 16), lambda i: (i, 0)),
        ],
        out_specs=[
            pl.BlockSpec((R, 16), lambda i: (i, 0)),
            pl.BlockSpec((1, 16), lambda i: (0, 0)),
            pl.BlockSpec((16, 16), lambda i: (0, 0)),
        ],
        out_shape=[
            jax.ShapeDtypeStruct((M, 16), jnp.float32),
            jax.ShapeDtypeStruct((1, 16), jnp.float32),
            jax.ShapeDtypeStruct((16, 16), jnp.float32),
        ],
        interpret=_INTERPRET,
    )(G, C)

    def full(shape):
        return pl.BlockSpec(shape, lambda i: tuple(0 for _ in shape))

    w_specs1 = [full((16, 32)), full((1, 32)), full((1, 32)), full((1, 32))]
    w_specs2 = [full((32, 32)), full((1, 32)), full((1, 32)), full((1, 32))]
    w_specs3 = [full((32, 64)), full((1, 64)), full((1, 64)), full((1, 64))]
    st16 = [full((1, 16)), full((16, 16))]
    st32 = [full((1, 32)), full((32, 32))]
    f_spec = pl.BlockSpec((R, 16), lambda i: (i, 0))

    zz32 = jnp.zeros((1, 32), jnp.float32)
    zg32 = jnp.zeros((32, 32), jnp.float32)

    s1, g1m = pl.pallas_call(
        functools.partial(_stats_body, 1, m_count=fM),
        grid=(nblk,),
        in_specs=[f_spec] + w_specs1 + w_specs2 + st16 + st32,
        out_specs=[full((1, 32)), full((32, 32))],
        out_shape=[
            jax.ShapeDtypeStruct((1, 32), jnp.float32),
            jax.ShapeDtypeStruct((32, 32), jnp.float32),
        ],
        interpret=_INTERPRET,
    )(f, w1t, b1, g1, be1, w2t, b2, g2, be2, sumf, gramf, zz32, zg32)

    s2, g2m = pl.pallas_call(
        functools.partial(_stats_body, 2, m_count=fM),
        grid=(nblk,),
        in_specs=[f_spec] + w_specs1 + w_specs2 + st16 + st32,
        out_specs=[full((1, 32)), full((32, 32))],
        out_shape=[
            jax.ShapeDtypeStruct((1, 32), jnp.float32),
            jax.ShapeDtypeStruct((32, 32), jnp.float32),
        ],
        interpret=_INTERPRET,
    )(f, w1t, b1, g1, be1, w2t, b2, g2, be2, sumf, gramf, s1, g1m)

    out = pl.pallas_call(
        functools.partial(_final_body, m_count=fM),
        grid=(nblk,),
        in_specs=([f_spec] + w_specs1 + w_specs2 + w_specs3
                  + st16 + st32 + st32),
        out_specs=pl.BlockSpec((rows, 64), lambda i: (i, 0)),
        out_shape=jax.ShapeDtypeStruct((M // _NSAMPLE, 64), jnp.float32),
        interpret=_INTERPRET,
    )(f, w1t, b1, g1, be1, w2t, b2, g2, be2, w3t, b3, g3, be3,
      sumf, gramf, s1, g1m, s2, g2m)
    return out


def kernel(xyz, points, W1, b1, g1, be1, W2, b2, g2, be2, W3, b3, g3, be3):
    B, N, _ = xyz.shape
    S = _NPOINT

    f0 = jax.random.randint(jax.random.key(1), (B,), 0, N).astype(jnp.int32)
    xyzT = jnp.transpose(xyz, (2, 0, 1))
    xyzT4 = xyzT.reshape(3, B, 8, N // 8)
    fps_flat = _run_fps(xyzT4, f0.reshape(1, B))
    fps_flat = jnp.transpose(fps_flat, (1, 0)).reshape(B * S)

    table = jnp.concatenate(
        [xyz, points, jnp.zeros((B, N, 7), jnp.float32)], axis=-1
    ).reshape(B * N, 16)

    c_rows = _gather_rows(table, fps_flat)
    new_xyz = c_rows[:, :3].reshape(B, S, 3)

    ball_idx = _run_select(
        c_rows.reshape(B, S, 16), jnp.transpose(xyz, (0, 2, 1)))
    g_rows = _gather_rows(table, ball_idx.reshape(B * S * _NSAMPLE))

    pad = jnp.zeros((7, 32), jnp.float32)
    w1t = jnp.concatenate([W1.T, pad], axis=0)
    out = _run_mlp(
        g_rows, c_rows, w1t,
        b1.reshape(1, 32), g1.reshape(1, 32), be1.reshape(1, 32),
        W2.T, b2.reshape(1, 32), g2.reshape(1, 32), be2.reshape(1, 32),
        W3.T, b3.reshape(1, 64), g3.reshape(1, 64), be3.reshape(1, 64))
    new_points = out.reshape(B, S, 64)
    return new_xyz, new_points


# speedup vs baseline: 18.6014x; 1.0017x over previous
import functools

import jax
import jax.numpy as jnp
from jax import lax
from jax.experimental import pallas as pl
from jax.experimental.pallas import tpu as pltpu
from jax.experimental.pallas import tpu_sc as plsc

_NPOINT = 1024
_NSAMPLE = 32
_RADIUS = 0.2
_INTERPRET = False

_BIGKEY = 0x7FFFFFFF
_IDXMASK = 8191
_KEYCLEAR = -8192


def _fps_body(xyzT_ref, f0_ref, idx_ref, dist_ref):
    _, B, SUB, NL = xyzT_ref.shape
    N = SUB * NL
    x0 = xyzT_ref[0]
    x1 = xyzT_ref[1]
    x2 = xyzT_ref[2]
    s_iota = lax.broadcasted_iota(jnp.int32, (B, SUB, NL), 1)
    l_iota = lax.broadcasted_iota(jnp.int32, (B, SUB, NL), 2)
    nflat = s_iota * NL + l_iota
    bcol = lax.broadcasted_iota(jnp.int32, (1, B), 1)
    dist_ref[...] = jnp.full((B, SUB, NL), 1e10, jnp.float32)

    def _redmax(v):
        return jnp.max(jnp.max(v, axis=2, keepdims=True), axis=1, keepdims=True)

    def _redmin(v):
        return jnp.min(jnp.min(v, axis=2, keepdims=True), axis=1, keepdims=True)

    def step(i, f):
        idx_ref[pl.ds(i, 1), :] = f.reshape(1, B) + bcol * N
        onehot = (nflat == f)[None]
        cc = jnp.max(jnp.max(jnp.where(onehot, xyzT_ref[...], -jnp.inf),
                             axis=3, keepdims=True), axis=2, keepdims=True)
        d = ((x0 - cc[0]) ** 2 + (x1 - cc[1]) ** 2 + (x2 - cc[2]) ** 2)
        dist = jnp.minimum(dist_ref[...], d)
        dist_ref[...] = dist
        m = _redmax(dist)
        cand = jnp.where(dist == m, nflat, N)
        return _redmin(cand)

    f0 = f0_ref[0, :].reshape(B, 1, 1)
    lax.fori_loop(0, _NPOINT, step, f0, unroll=False)


def _run_fps(xyzT, f0):
    B = xyzT.shape[1]
    return pl.pallas_call(
        _fps_body,
        out_shape=jax.ShapeDtypeStruct((_NPOINT, B), jnp.int32),
        scratch_shapes=[pltpu.VMEM(xyzT.shape[1:], jnp.float32)],
        interpret=_INTERPRET,
    )(xyzT, f0)


def _select_body(q_ref, xyzT_ref, out_ref, key_ref):
    SB = q_ref.shape[1]
    N = xyzT_ref.shape[2]
    b = pl.program_id(0)
    x0 = xyzT_ref[0, 0:1, :]
    x1 = xyzT_ref[0, 1:2, :]
    x2 = xyzT_ref[0, 2:3, :]
    q0 = q_ref[0, :, 0:1]
    q1 = q_ref[0, :, 1:2]
    q2 = q_ref[0, :, 2:3]
    x2sum = x0 * x0 + x1 * x1 + x2 * x2
    q2sum = q0 * q0 + q1 * q1 + q2 * q2

    def rne16(v):
        bv = lax.bitcast_convert_type(v, jnp.int32)
        r = bv + 0x7FFF + jnp.bitwise_and(lax.shift_right_arithmetic(bv, 16), 1)
        return lax.bitcast_convert_type(jnp.bitwise_and(r, -65536), jnp.float32)

    acc = (rne16(q0) * rne16(x0) + rne16(q1) * rne16(x1)
           + rne16(q2) * rne16(x2))
    d2 = (q2sum + x2sum) - 2.0 * acc
    d = jnp.sqrt(jnp.maximum(d2, 0.0))
    n_iota = lax.broadcasted_iota(jnp.int32, (SB, N), 1)
    bits = lax.bitcast_convert_type(d, jnp.int32)
    key_all = jnp.bitwise_or(jnp.bitwise_and(bits, _KEYCLEAR), n_iota)
    nearest = jnp.bitwise_and(
        jnp.min(key_all, axis=1, keepdims=True), _IDXMASK)
    key = jnp.where(d <= jnp.float32(_RADIUS), key_all, _BIGKEY)
    kmin0 = jnp.min(key, axis=1, keepdims=True)
    key_ref[...] = key
    col = lax.broadcasted_iota(jnp.int32, (SB, _NSAMPLE), 1)
    out0 = jnp.zeros((SB, _NSAMPLE), jnp.int32)

    def step(k, carry):
        out, kmin = carry
        sel = jnp.where(kmin != _BIGKEY,
                        jnp.bitwise_and(kmin, _IDXMASK), nearest)
        out = jnp.where(col == k, sel, out)
        kk = key_ref[...]
        upd = jnp.where(kk == kmin, _BIGKEY, kk)
        key_ref[...] = upd
        return out, jnp.min(upd, axis=1, keepdims=True)

    out, _ = lax.fori_loop(0, _NSAMPLE, step, (out0, kmin0), unroll=False)
    out_ref[0] = out + b * N


def _run_select(q_rows, xyzT2, SB=256):
    B, S, _ = q_rows.shape
    N = xyzT2.shape[2]
    return pl.pallas_call(
        _select_body,
        grid=(B, S // SB),
        in_specs=[
            pl.BlockSpec((1, SB, 16), lambda b, s: (b, s, 0)),
            pl.BlockSpec((1, 3, N), lambda b, s: (b, 0, 0)),
        ],
        out_specs=pl.BlockSpec((1, SB, _NSAMPLE), lambda b, s: (b, s, 0)),
        out_shape=jax.ShapeDtypeStruct((B, S, _NSAMPLE), jnp.int32),
        scratch_shapes=[pltpu.VMEM((SB, N), jnp.int32)],
        interpret=_INTERPRET,
    )(q_rows, xyzT2)


def _gather_rows(table, idx):
    M = idx.shape[0]
    D = table.shape[1]
    NW = 32
    rpw = M // NW
    CH = min(128, rpw)
    nch = rpw // CH
    mesh = plsc.VectorSubcoreMesh(core_axis_name="c", subcore_axis_name="s")
    idx2 = idx.reshape(M // CH, CH)

    @functools.partial(
        pl.kernel,
        out_type=jax.ShapeDtypeStruct((M, D), jnp.float32),
        mesh=mesh,
        scratch_types=[
            pltpu.VMEM((nch, CH), jnp.int32),
            pltpu.VMEM((CH, D), jnp.float32),
            pltpu.SemaphoreType.DMA,
        ],
        compiler_params=pltpu.CompilerParams(use_tc_tiling_on_sc=False),
    )
    def k(table_hbm, idx_hbm, out_hbm, idx_v, rows_v, sem):
        wid = lax.axis_index("s") * 2 + lax.axis_index("c")
        base = wid * rpw
        pltpu.sync_copy(idx_hbm.at[pl.ds(wid * nch, nch)], idx_v)

        def chunk(j, _):
            pltpu.async_copy(table_hbm.at[idx_v.at[j]], rows_v, sem).wait()
            pltpu.sync_copy(rows_v, out_hbm.at[pl.ds(base + j * CH, CH)])
            return 0

        lax.fori_loop(0, nch, chunk, 0, unroll=False)

    return k(table, idx2)


_EPS = 1e-5

def _mxu_dot(a, b):
    return jnp.dot(a.astype(jnp.bfloat16), b.astype(jnp.bfloat16),
                   preferred_element_type=jnp.float32)


def _bn_affine(mean_in, gram_in, wt, bvec, g, be, m_count):
    mu_in = mean_in / m_count
    mu = jnp.dot(mu_in, wt, preferred_element_type=jnp.float32) + bvec
    t = jnp.dot(gram_in / m_count, wt, preferred_element_type=jnp.float32)
    ex2 = (jnp.sum(wt * t, axis=0, keepdims=True)
           + 2.0 * bvec * (mu - bvec) + bvec * bvec)
    var = ex2 - mu * mu
    scale = g / jnp.sqrt(var + _EPS)
    shift = be - mu * scale
    return scale, shift


def _k1_body(g_ref, c_ref, f_ref, sum_ref, gram_ref):
    R = g_ref.shape[0]
    rows = R // _NSAMPLE
    c = c_ref[...]
    cmask = jnp.where(
        lax.broadcasted_iota(jnp.int32, c.shape, 1) < 3, c, 0.0)
    f = (g_ref[...].reshape(rows, _NSAMPLE, 16) - cmask[:, None, :])
    f2 = f.reshape(R, 16)
    f_ref[...] = f2

    @pl.when(pl.program_id(0) == 0)
    def _():
        sum_ref[...] = jnp.zeros_like(sum_ref)
        gram_ref[...] = jnp.zeros_like(gram_ref)

    sum_ref[...] += jnp.sum(f2, axis=0, keepdims=True)
    gram_ref[...] += lax.dot_general(
        f2, f2, (((0,), (0,)), ((), ())),
        preferred_element_type=jnp.float32)


def _stats_body(nlayer, f_ref, w1_ref, b1_ref, g1_ref, be1_ref,
                w2_ref, b2_ref, g2_ref, be2_ref,
                sumf_ref, gramf_ref, s1_ref, g1m_ref,
                sy_ref, gy_ref, m_count=None):
    y = f_ref[...]
    s, t = _bn_affine(sumf_ref[...], gramf_ref[...], w1_ref[...],
                      b1_ref[...], g1_ref[...], be1_ref[...], m_count)
    y = jnp.maximum(
        _mxu_dot(y, w1_ref[...]) * s + t,
        0.0)
    if nlayer == 2:
        s, t = _bn_affine(s1_ref[...], g1m_ref[...], w2_ref[...],
                          b2_ref[...], g2_ref[...], be2_ref[...], m_count)
        y = jnp.maximum(
            _mxu_dot(y, w2_ref[...]) * s
            + t, 0.0)

    @pl.when(pl.program_id(0) == 0)
    def _():
        sy_ref[...] = jnp.zeros_like(sy_ref)
        gy_ref[...] = jnp.zeros_like(gy_ref)

    sy_ref[...] += jnp.sum(y, axis=0, keepdims=True)
    gy_ref[...] += lax.dot_general(
        y, y, (((0,), (0,)), ((), ())), preferred_element_type=jnp.float32)


def _final_body(f_ref, w1_ref, b1_ref, g1_ref, be1_ref,
                w2_ref, b2_ref, g2_ref, be2_ref,
                w3_ref, b3_ref, g3_ref, be3_ref,
                sumf_ref, gramf_ref, s1_ref, g1m_ref, s2_ref, g2m_ref,
                out_ref, m_count=None):
    y = f_ref[...]
    s, t = _bn_affine(sumf_ref[...], gramf_ref[...], w1_ref[...],
                      b1_ref[...], g1_ref[...], be1_ref[...], m_count)
    y = jnp.maximum(
        _mxu_dot(y, w1_ref[...]) * s + t,
        0.0)
    s, t = _bn_affine(s1_ref[...], g1m_ref[...], w2_ref[...],
                      b2_ref[...], g2_ref[...], be2_ref[...], m_count)
    y = jnp.maximum(
        _mxu_dot(y, w2_ref[...]) * s + t,
        0.0)
    s, t = _bn_affine(s2_ref[...], g2m_ref[...], w3_ref[...],
                      b3_ref[...], g3_ref[...], be3_ref[...], m_count)
    y = jnp.maximum(
        _mxu_dot(y, w3_ref[...]) * s + t,
        0.0)
    R = y.shape[0]
    out_ref[...] = jnp.max(
        y.reshape(R // _NSAMPLE, _NSAMPLE, y.shape[1]), axis=1)


def _run_mlp(G, C, w1t, b1, g1, be1, w2t, b2, g2, be2, w3t, b3, g3, be3):
    M = G.shape[0]
    R = 8192
    nblk = M // R
    rows = R // _NSAMPLE
    fM = float(M)

    f, sumf, gramf = pl.pallas_call(
        _k1_body,
        grid=(nblk,),
        in_specs=[
            pl.BlockSpec((R, 16), lambda i: (i, 0)),
            pl.BlockSpec((rows, 16), lambda i: (i, 0)),
        ],
        out_specs=[
            pl.BlockSpec((R, 16), lambda i: (i, 0)),
            pl.BlockSpec((1, 16), lambda i: (0, 0)),
            pl.BlockSpec((16, 16), lambda i: (0, 0)),
        ],
        out_shape=[
            jax.ShapeDtypeStruct((M, 16), jnp.float32),
            jax.ShapeDtypeStruct((1, 16), jnp.float32),
            jax.ShapeDtypeStruct((16, 16), jnp.float32),
        ],
        interpret=_INTERPRET,
    )(G, C)

    def full(shape):
        return pl.BlockSpec(shape, lambda i: tuple(0 for _ in shape))

    w_specs1 = [full((16, 32)), full((1, 32)), full((1, 32)), full((1, 32))]
    w_specs2 = [full((32, 32)), full((1, 32)), full((1, 32)), full((1, 32))]
    w_specs3 = [full((32, 64)), full((1, 64)), full((1, 64)), full((1, 64))]
    st16 = [full((1, 16)), full((16, 16))]
    st32 = [full((1, 32)), full((32, 32))]
    f_spec = pl.BlockSpec((R, 16), lambda i: (i, 0))

    zz32 = jnp.zeros((1, 32), jnp.float32)
    zg32 = jnp.zeros((32, 32), jnp.float32)

    s1, g1m = pl.pallas_call(
        functools.partial(_stats_body, 1, m_count=fM),
        grid=(nblk,),
        in_specs=[f_spec] + w_specs1 + w_specs2 + st16 + st32,
        out_specs=[full((1, 32)), full((32, 32))],
        out_shape=[
            jax.ShapeDtypeStruct((1, 32), jnp.float32),
            jax.ShapeDtypeStruct((32, 32), jnp.float32),
        ],
        interpret=_INTERPRET,
    )(f, w1t, b1, g1, be1, w2t, b2, g2, be2, sumf, gramf, zz32, zg32)

    s2, g2m = pl.pallas_call(
        functools.partial(_stats_body, 2, m_count=fM),
        grid=(nblk,),
        in_specs=[f_spec] + w_specs1 + w_specs2 + st16 + st32,
        out_specs=[full((1, 32)), full((32, 32))],
        out_shape=[
            jax.ShapeDtypeStruct((1, 32), jnp.float32),
            jax.ShapeDtypeStruct((32, 32), jnp.float32),
        ],
        interpret=_INTERPRET,
    )(f, w1t, b1, g1, be1, w2t, b2, g2, be2, sumf, gramf, s1, g1m)

    out = pl.pallas_call(
        functools.partial(_final_body, m_count=fM),
        grid=(nblk,),
        in_specs=([f_spec] + w_specs1 + w_specs2 + w_specs3
                  + st16 + st32 + st32),
        out_specs=pl.BlockSpec((rows, 64), lambda i: (i, 0)),
        out_shape=jax.ShapeDtypeStruct((M // _NSAMPLE, 64), jnp.float32),
        interpret=_INTERPRET,
    )(f, w1t, b1, g1, be1, w2t, b2, g2, be2, w3t, b3, g3, be3,
      sumf, gramf, s1, g1m, s2, g2m)
    return out


def kernel(xyz, points, W1, b1, g1, be1, W2, b2, g2, be2, W3, b3, g3, be3):
    B, N, _ = xyz.shape
    S = _NPOINT

    f0 = jax.random.randint(jax.random.key(1), (B,), 0, N).astype(jnp.int32)
    xyzT = jnp.transpose(xyz, (2, 0, 1))
    xyzT4 = xyzT.reshape(3, B, 8, N // 8)
    fps_flat = _run_fps(xyzT4, f0.reshape(1, B))
    fps_flat = jnp.transpose(fps_flat, (1, 0)).reshape(B * S)

    table = jnp.concatenate(
        [xyz, points, jnp.zeros((B, N, 7), jnp.float32)], axis=-1
    ).reshape(B * N, 16)

    c_rows = _gather_rows(table, fps_flat)
    new_xyz = c_rows[:, :3].reshape(B, S, 3)

    ball_idx = _run_select(
        c_rows.reshape(B, S, 16), jnp.transpose(xyz, (0, 2, 1)))
    g_rows = _gather_rows(table, ball_idx.reshape(B * S * _NSAMPLE))

    pad = jnp.zeros((7, 32), jnp.float32)
    w1t = jnp.concatenate([W1.T, pad], axis=0)
    out = _run_mlp(
        g_rows, c_rows, w1t,
        b1.reshape(1, 32), g1.reshape(1, 32), be1.reshape(1, 32),
        W2.T, b2.reshape(1, 32), g2.reshape(1, 32), be2.reshape(1, 32),
        W3.T, b3.reshape(1, 64), g3.reshape(1, 64), be3.reshape(1, 64))
    new_points = out.reshape(B, S, 64)
    return new_xyz, new_points


# speedup vs baseline: 20.1801x; 1.0849x over previous
import functools

import jax
import jax.numpy as jnp
from jax import lax
from jax.experimental import pallas as pl
from jax.experimental.pallas import tpu as pltpu
from jax.experimental.pallas import tpu_sc as plsc

_NPOINT = 1024
_NSAMPLE = 32
_RADIUS = 0.2
_INTERPRET = False

_BIGKEY = 0x7FFFFFFF
_IDXMASK = 8191
_KEYCLEAR = -8192


def _fps_body(xyzT_ref, f0_ref, idx_ref, dist_ref):
    _, B, SUB, NL = xyzT_ref.shape
    N = SUB * NL
    x0 = xyzT_ref[0]
    x1 = xyzT_ref[1]
    x2 = xyzT_ref[2]
    s_iota = lax.broadcasted_iota(jnp.int32, (B, SUB, NL), 1)
    l_iota = lax.broadcasted_iota(jnp.int32, (B, SUB, NL), 2)
    nflat = s_iota * NL + l_iota
    bcol = lax.broadcasted_iota(jnp.int32, (1, B), 1)
    dist_ref[...] = jnp.full((B, SUB, NL), 1e10, jnp.float32)

    def _redmax(v):
        return jnp.max(jnp.max(v, axis=2, keepdims=True), axis=1, keepdims=True)

    def _redmin(v):
        return jnp.min(jnp.min(v, axis=2, keepdims=True), axis=1, keepdims=True)

    def step(i, f):
        idx_ref[pl.ds(i, 1), :] = f.reshape(1, B) + bcol * N
        onehot = (nflat == f)[None]
        cc = jnp.max(jnp.max(jnp.where(onehot, xyzT_ref[...], -jnp.inf),
                             axis=3, keepdims=True), axis=2, keepdims=True)
        d = ((x0 - cc[0]) ** 2 + (x1 - cc[1]) ** 2 + (x2 - cc[2]) ** 2)
        dist = jnp.minimum(dist_ref[...], d)
        dist_ref[...] = dist
        m = _redmax(dist)
        cand = jnp.where(dist == m, nflat, N)
        return _redmin(cand)

    f0 = f0_ref[0, :].reshape(B, 1, 1)
    lax.fori_loop(0, _NPOINT, step, f0, unroll=False)


def _run_fps(xyzT, f0):
    B = xyzT.shape[1]
    return pl.pallas_call(
        _fps_body,
        out_shape=jax.ShapeDtypeStruct((_NPOINT, B), jnp.int32),
        scratch_shapes=[pltpu.VMEM(xyzT.shape[1:], jnp.float32)],
        interpret=_INTERPRET,
    )(xyzT, f0)


def _select_body(q_ref, xyzT_ref, out_ref, key_ref):
    SB = q_ref.shape[1]
    N = xyzT_ref.shape[2]
    b = pl.program_id(0)
    x0 = xyzT_ref[0, 0:1, :]
    x1 = xyzT_ref[0, 1:2, :]
    x2 = xyzT_ref[0, 2:3, :]
    q0 = q_ref[0, :, 0:1]
    q1 = q_ref[0, :, 1:2]
    q2 = q_ref[0, :, 2:3]
    x2sum = x0 * x0 + x1 * x1 + x2 * x2
    q2sum = q0 * q0 + q1 * q1 + q2 * q2

    def rne16(v):
        bv = lax.bitcast_convert_type(v, jnp.int32)
        r = bv + 0x7FFF + jnp.bitwise_and(lax.shift_right_arithmetic(bv, 16), 1)
        return lax.bitcast_convert_type(jnp.bitwise_and(r, -65536), jnp.float32)

    acc = (rne16(q0) * rne16(x0) + rne16(q1) * rne16(x1)
           + rne16(q2) * rne16(x2))
    d2 = (q2sum + x2sum) - 2.0 * acc
    d = jnp.sqrt(jnp.maximum(d2, 0.0))
    n_iota = lax.broadcasted_iota(jnp.int32, (SB, N), 1)
    bits = lax.bitcast_convert_type(d, jnp.int32)
    key_all = jnp.bitwise_or(jnp.bitwise_and(bits, _KEYCLEAR), n_iota)
    nearest = jnp.bitwise_and(
        jnp.min(key_all, axis=1, keepdims=True), _IDXMASK)
    key = jnp.where(d <= jnp.float32(_RADIUS), key_all, _BIGKEY)
    key_ref[...] = key
    col = lax.broadcasted_iota(jnp.int32, (SB, _NSAMPLE), 1)
    out0 = jnp.zeros((SB, _NSAMPLE), jnp.int32)
    kprev0 = jnp.full((SB, 1), -1, jnp.int32)

    def step(k, carry):
        out, kprev = carry
        kk = key_ref[...]
        kmin = jnp.min(jnp.where(kk > kprev, kk, _BIGKEY),
                       axis=1, keepdims=True)
        sel = jnp.where(kmin != _BIGKEY,
                        jnp.bitwise_and(kmin, _IDXMASK), nearest)
        out = jnp.where(col == k, sel, out)
        return out, kmin

    out, _ = lax.fori_loop(0, _NSAMPLE, step, (out0, kprev0), unroll=False)
    out_ref[0] = out + b * N


def _run_select(q_rows, xyzT2, SB=256):
    B, S, _ = q_rows.shape
    N = xyzT2.shape[2]
    return pl.pallas_call(
        _select_body,
        grid=(B, S // SB),
        in_specs=[
            pl.BlockSpec((1, SB, 16), lambda b, s: (b, s, 0)),
            pl.BlockSpec((1, 3, N), lambda b, s: (b, 0, 0)),
        ],
        out_specs=pl.BlockSpec((1, SB, _NSAMPLE), lambda b, s: (b, s, 0)),
        out_shape=jax.ShapeDtypeStruct((B, S, _NSAMPLE), jnp.int32),
        scratch_shapes=[pltpu.VMEM((SB, N), jnp.int32)],
        interpret=_INTERPRET,
    )(q_rows, xyzT2)


def _gather_rows(table, idx):
    M = idx.shape[0]
    D = table.shape[1]
    NW = 32
    rpw = M // NW
    CH = min(128, rpw)
    nch = rpw // CH
    mesh = plsc.VectorSubcoreMesh(core_axis_name="c", subcore_axis_name="s")
    idx2 = idx.reshape(M // CH, CH)

    @functools.partial(
        pl.kernel,
        out_type=jax.ShapeDtypeStruct((M, D), jnp.float32),
        mesh=mesh,
        scratch_types=[
            pltpu.VMEM((nch, CH), jnp.int32),
            pltpu.VMEM((CH, D), jnp.float32),
            pltpu.SemaphoreType.DMA,
        ],
        compiler_params=pltpu.CompilerParams(use_tc_tiling_on_sc=False),
    )
    def k(table_hbm, idx_hbm, out_hbm, idx_v, rows_v, sem):
        wid = lax.axis_index("s") * 2 + lax.axis_index("c")
        base = wid * rpw
        pltpu.sync_copy(idx_hbm.at[pl.ds(wid * nch, nch)], idx_v)

        def chunk(j, _):
            pltpu.async_copy(table_hbm.at[idx_v.at[j]], rows_v, sem).wait()
            pltpu.sync_copy(rows_v, out_hbm.at[pl.ds(base + j * CH, CH)])
            return 0

        lax.fori_loop(0, nch, chunk, 0, unroll=False)

    return k(table, idx2)


_EPS = 1e-5

def _mxu_dot(a, b):
    return jnp.dot(a.astype(jnp.bfloat16), b.astype(jnp.bfloat16),
                   preferred_element_type=jnp.float32)


def _bn_affine(mean_in, gram_in, wt, bvec, g, be, m_count):
    mu_in = mean_in / m_count
    mu = jnp.dot(mu_in, wt, preferred_element_type=jnp.float32) + bvec
    t = jnp.dot(gram_in / m_count, wt, preferred_element_type=jnp.float32)
    ex2 = (jnp.sum(wt * t, axis=0, keepdims=True)
           + 2.0 * bvec * (mu - bvec) + bvec * bvec)
    var = ex2 - mu * mu
    scale = g / jnp.sqrt(var + _EPS)
    shift = be - mu * scale
    return scale, shift


def _k1_body(g_ref, c_ref, f_ref, sum_ref, gram_ref):
    R = g_ref.shape[0]
    rows = R // _NSAMPLE
    c = c_ref[...]
    cmask = jnp.where(
        lax.broadcasted_iota(jnp.int32, c.shape, 1) < 3, c, 0.0)
    f = (g_ref[...].reshape(rows, _NSAMPLE, 16) - cmask[:, None, :])
    f2 = f.reshape(R, 16)
    f_ref[...] = f2

    @pl.when(pl.program_id(0) == 0)
    def _():
        sum_ref[...] = jnp.zeros_like(sum_ref)
        gram_ref[...] = jnp.zeros_like(gram_ref)

    sum_ref[...] += jnp.sum(f2, axis=0, keepdims=True)
    gram_ref[...] += lax.dot_general(
        f2, f2, (((0,), (0,)), ((), ())),
        preferred_element_type=jnp.float32)


def _stats_body(nlayer, f_ref, w1_ref, b1_ref, g1_ref, be1_ref,
                w2_ref, b2_ref, g2_ref, be2_ref,
                sumf_ref, gramf_ref, s1_ref, g1m_ref,
                sy_ref, gy_ref, m_count=None):
    y = f_ref[...]
    s, t = _bn_affine(sumf_ref[...], gramf_ref[...], w1_ref[...],
                      b1_ref[...], g1_ref[...], be1_ref[...], m_count)
    y = jnp.maximum(
        _mxu_dot(y, w1_ref[...]) * s + t,
        0.0)
    if nlayer == 2:
        s, t = _bn_affine(s1_ref[...], g1m_ref[...], w2_ref[...],
                          b2_ref[...], g2_ref[...], be2_ref[...], m_count)
        y = jnp.maximum(
            _mxu_dot(y, w2_ref[...]) * s
            + t, 0.0)

    @pl.when(pl.program_id(0) == 0)
    def _():
        sy_ref[...] = jnp.zeros_like(sy_ref)
        gy_ref[...] = jnp.zeros_like(gy_ref)

    sy_ref[...] += jnp.sum(y, axis=0, keepdims=True)
    gy_ref[...] += lax.dot_general(
        y, y, (((0,), (0,)), ((), ())), preferred_element_type=jnp.float32)


def _final_body(f_ref, w1_ref, b1_ref, g1_ref, be1_ref,
                w2_ref, b2_ref, g2_ref, be2_ref,
                w3_ref, b3_ref, g3_ref, be3_ref,
                sumf_ref, gramf_ref, s1_ref, g1m_ref, s2_ref, g2m_ref,
                out_ref, m_count=None):
    y = f_ref[...]
    s, t = _bn_affine(sumf_ref[...], gramf_ref[...], w1_ref[...],
                      b1_ref[...], g1_ref[...], be1_ref[...], m_count)
    y = jnp.maximum(
        _mxu_dot(y, w1_ref[...]) * s + t,
        0.0)
    s, t = _bn_affine(s1_ref[...], g1m_ref[...], w2_ref[...],
                      b2_ref[...], g2_ref[...], be2_ref[...], m_count)
    y = jnp.maximum(
        _mxu_dot(y, w2_ref[...]) * s + t,
        0.0)
    s, t = _bn_affine(s2_ref[...], g2m_ref[...], w3_ref[...],
                      b3_ref[...], g3_ref[...], be3_ref[...], m_count)
    y = jnp.maximum(
        _mxu_dot(y, w3_ref[...]) * s + t,
        0.0)
    R = y.shape[0]
    out_ref[...] = jnp.max(
        y.reshape(R // _NSAMPLE, _NSAMPLE, y.shape[1]), axis=1)


def _run_mlp(G, C, w1t, b1, g1, be1, w2t, b2, g2, be2, w3t, b3, g3, be3):
    M = G.shape[0]
    R = 8192
    nblk = M // R
    rows = R // _NSAMPLE
    fM = float(M)

    f, sumf, gramf = pl.pallas_call(
        _k1_body,
        grid=(nblk,),
        in_specs=[
            pl.BlockSpec((R, 16), lambda i: (i, 0)),
            pl.BlockSpec((rows, 16), lambda i: (i, 0)),
        ],
        out_specs=[
            pl.BlockSpec((R, 16), lambda i: (i, 0)),
            pl.BlockSpec((1, 16), lambda i: (0, 0)),
            pl.BlockSpec((16, 16), lambda i: (0, 0)),
        ],
        out_shape=[
            jax.ShapeDtypeStruct((M, 16), jnp.float32),
            jax.ShapeDtypeStruct((1, 16), jnp.float32),
            jax.ShapeDtypeStruct((16, 16), jnp.float32),
        ],
        interpret=_INTERPRET,
    )(G, C)

    def full(shape):
        return pl.BlockSpec(shape, lambda i: tuple(0 for _ in shape))

    w_specs1 = [full((16, 32)), full((1, 32)), full((1, 32)), full((1, 32))]
    w_specs2 = [full((32, 32)), full((1, 32)), full((1, 32)), full((1, 32))]
    w_specs3 = [full((32, 64)), full((1, 64)), full((1, 64)), full((1, 64))]
    st16 = [full((1, 16)), full((16, 16))]
    st32 = [full((1, 32)), full((32, 32))]
    f_spec = pl.BlockSpec((R, 16), lambda i: (i, 0))

    zz32 = jnp.zeros((1, 32), jnp.float32)
    zg32 = jnp.zeros((32, 32), jnp.float32)

    s1, g1m = pl.pallas_call(
        functools.partial(_stats_body, 1, m_count=fM),
        grid=(nblk,),
        in_specs=[f_spec] + w_specs1 + w_specs2 + st16 + st32,
        out_specs=[full((1, 32)), full((32, 32))],
        out_shape=[
            jax.ShapeDtypeStruct((1, 32), jnp.float32),
            jax.ShapeDtypeStruct((32, 32), jnp.float32),
        ],
        interpret=_INTERPRET,
    )(f, w1t, b1, g1, be1, w2t, b2, g2, be2, sumf, gramf, zz32, zg32)

    s2, g2m = pl.pallas_call(
        functools.partial(_stats_body, 2, m_count=fM),
        grid=(nblk,),
        in_specs=[f_spec] + w_specs1 + w_specs2 + st16 + st32,
        out_specs=[full((1, 32)), full((32, 32))],
        out_shape=[
            jax.ShapeDtypeStruct((1, 32), jnp.float32),
            jax.ShapeDtypeStruct((32, 32), jnp.float32),
        ],
        interpret=_INTERPRET,
    )(f, w1t, b1, g1, be1, w2t, b2, g2, be2, sumf, gramf, s1, g1m)

    out = pl.pallas_call(
        functools.partial(_final_body, m_count=fM),
        grid=(nblk,),
        in_specs=([f_spec] + w_specs1 + w_specs2 + w_specs3
                  + st16 + st32 + st32),
        out_specs=pl.BlockSpec((rows, 64), lambda i: (i, 0)),
        out_shape=jax.ShapeDtypeStruct((M // _NSAMPLE, 64), jnp.float32),
        interpret=_INTERPRET,
    )(f, w1t, b1, g1, be1, w2t, b2, g2, be2, w3t, b3, g3, be3,
      sumf, gramf, s1, g1m, s2, g2m)
    return out


def kernel(xyz, points, W1, b1, g1, be1, W2, b2, g2, be2, W3, b3, g3, be3):
    B, N, _ = xyz.shape
    S = _NPOINT

    f0 = jax.random.randint(jax.random.key(1), (B,), 0, N).astype(jnp.int32)
    xyzT = jnp.transpose(xyz, (2, 0, 1))
    xyzT4 = xyzT.reshape(3, B, 8, N // 8)
    fps_flat = _run_fps(xyzT4, f0.reshape(1, B))
    fps_flat = jnp.transpose(fps_flat, (1, 0)).reshape(B * S)

    table = jnp.concatenate(
        [xyz, points, jnp.zeros((B, N, 7), jnp.float32)], axis=-1
    ).reshape(B * N, 16)

    c_rows = _gather_rows(table, fps_flat)
    new_xyz = c_rows[:, :3].reshape(B, S, 3)

    ball_idx = _run_select(
        c_rows.reshape(B, S, 16), jnp.transpose(xyz, (0, 2, 1)))
    g_rows = _gather_rows(table, ball_idx.reshape(B * S * _NSAMPLE))

    pad = jnp.zeros((7, 32), jnp.float32)
    w1t = jnp.concatenate([W1.T, pad], axis=0)
    out = _run_mlp(
        g_rows, c_rows, w1t,
        b1.reshape(1, 32), g1.reshape(1, 32), be1.reshape(1, 32),
        W2.T, b2.reshape(1, 32), g2.reshape(1, 32), be2.reshape(1, 32),
        W3.T, b3.reshape(1, 64), g3.reshape(1, 64), be3.reshape(1, 64))
    new_points = out.reshape(B, S, 64)
    return new_xyz, new_points


# speedup vs baseline: 20.4839x; 1.0151x over previous
import functools

import jax
import jax.numpy as jnp
from jax import lax
from jax.experimental import pallas as pl
from jax.experimental.pallas import tpu as pltpu
from jax.experimental.pallas import tpu_sc as plsc

_NPOINT = 1024
_NSAMPLE = 32
_RADIUS = 0.2
_INTERPRET = False

_BIGKEY = 0x7FFFFFFF
_IDXMASK = 8191
_KEYCLEAR = -8192


def _fps_body(xyzT_ref, f0_ref, idx_ref):
    _, B, SUB, NL = xyzT_ref.shape
    N = SUB * NL
    x0 = xyzT_ref[0]
    x1 = xyzT_ref[1]
    x2 = xyzT_ref[2]
    s_iota = lax.broadcasted_iota(jnp.int32, (B, SUB, NL), 1)
    l_iota = lax.broadcasted_iota(jnp.int32, (B, SUB, NL), 2)
    nflat = s_iota * NL + l_iota
    bcol = lax.broadcasted_iota(jnp.int32, (1, B), 1)

    def _redmax(v):
        return jnp.max(jnp.max(v, axis=2, keepdims=True), axis=1, keepdims=True)

    def _redmin(v):
        return jnp.min(jnp.min(v, axis=2, keepdims=True), axis=1, keepdims=True)

    def step(i, carry):
        f, dist = carry
        idx_ref[pl.ds(i, 1), :] = f.reshape(1, B) + bcol * N
        onehot = (nflat == f)[None]
        cc = jnp.max(jnp.max(jnp.where(onehot, xyzT_ref[...], -jnp.inf),
                             axis=3, keepdims=True), axis=2, keepdims=True)
        d = ((x0 - cc[0]) ** 2 + (x1 - cc[1]) ** 2 + (x2 - cc[2]) ** 2)
        dist = jnp.minimum(dist, d)
        m = _redmax(dist)
        cand = jnp.where(dist == m, nflat, N)
        return _redmin(cand), dist

    f0 = f0_ref[0, :].reshape(B, 1, 1)
    dist0 = jnp.full((B, SUB, NL), 1e10, jnp.float32)
    lax.fori_loop(0, _NPOINT, step, (f0, dist0), unroll=2)


def _run_fps(xyzT, f0):
    B = xyzT.shape[1]
    return pl.pallas_call(
        _fps_body,
        out_shape=jax.ShapeDtypeStruct((_NPOINT, B), jnp.int32),
        interpret=_INTERPRET,
    )(xyzT, f0)


def _select_body(q_ref, xyzT_ref, out_ref, key_ref):
    SB = q_ref.shape[1]
    N = xyzT_ref.shape[2]
    b = pl.program_id(0)
    x0 = xyzT_ref[0, 0:1, :]
    x1 = xyzT_ref[0, 1:2, :]
    x2 = xyzT_ref[0, 2:3, :]
    q0 = q_ref[0, :, 0:1]
    q1 = q_ref[0, :, 1:2]
    q2 = q_ref[0, :, 2:3]
    x2sum = x0 * x0 + x1 * x1 + x2 * x2
    q2sum = q0 * q0 + q1 * q1 + q2 * q2

    def rne16(v):
        bv = lax.bitcast_convert_type(v, jnp.int32)
        r = bv + 0x7FFF + jnp.bitwise_and(lax.shift_right_arithmetic(bv, 16), 1)
        return lax.bitcast_convert_type(jnp.bitwise_and(r, -65536), jnp.float32)

    acc = (rne16(q0) * rne16(x0) + rne16(q1) * rne16(x1)
           + rne16(q2) * rne16(x2))
    d2 = (q2sum + x2sum) - 2.0 * acc
    d = jnp.sqrt(jnp.maximum(d2, 0.0))
    n_iota = lax.broadcasted_iota(jnp.int32, (SB, N), 1)
    bits = lax.bitcast_convert_type(d, jnp.int32)
    key_all = jnp.bitwise_or(jnp.bitwise_and(bits, _KEYCLEAR), n_iota)
    nearest = jnp.bitwise_and(
        jnp.min(key_all, axis=1, keepdims=True), _IDXMASK)
    key = jnp.where(d <= jnp.float32(_RADIUS), key_all, _BIGKEY)
    key_ref[...] = key
    col = lax.broadcasted_iota(jnp.int32, (SB, _NSAMPLE), 1)
    out0 = jnp.zeros((SB, _NSAMPLE), jnp.int32)
    kprev0 = jnp.full((SB, 1), -1, jnp.int32)

    def step(k, carry):
        out, kprev = carry
        kk = key_ref[...]
        kmin = jnp.min(jnp.where(kk > kprev, kk, _BIGKEY),
                       axis=1, keepdims=True)
        sel = jnp.where(kmin != _BIGKEY,
                        jnp.bitwise_and(kmin, _IDXMASK), nearest)
        out = jnp.where(col == k, sel, out)
        return out, kmin

    out, _ = lax.fori_loop(0, _NSAMPLE, step, (out0, kprev0), unroll=False)
    out_ref[0] = out + b * N


def _run_select(q_rows, xyzT2, SB=256):
    B, S, _ = q_rows.shape
    N = xyzT2.shape[2]
    return pl.pallas_call(
        _select_body,
        grid=(B, S // SB),
        in_specs=[
            pl.BlockSpec((1, SB, 16), lambda b, s: (b, s, 0)),
            pl.BlockSpec((1, 3, N), lambda b, s: (b, 0, 0)),
        ],
        out_specs=pl.BlockSpec((1, SB, _NSAMPLE), lambda b, s: (b, s, 0)),
        out_shape=jax.ShapeDtypeStruct((B, S, _NSAMPLE), jnp.int32),
        scratch_shapes=[pltpu.VMEM((SB, N), jnp.int32)],
        interpret=_INTERPRET,
    )(q_rows, xyzT2)


def _gather_rows(table, idx):
    M = idx.shape[0]
    D = table.shape[1]
    NW = 32
    rpw = M // NW
    CH = min(128, rpw)
    nch = rpw // CH
    mesh = plsc.VectorSubcoreMesh(core_axis_name="c", subcore_axis_name="s")
    idx2 = idx.reshape(M // CH, CH)

    @functools.partial(
        pl.kernel,
        out_type=jax.ShapeDtypeStruct((M, D), jnp.float32),
        mesh=mesh,
        scratch_types=[
            pltpu.VMEM((nch, CH), jnp.int32),
            pltpu.VMEM((CH, D), jnp.float32),
            pltpu.SemaphoreType.DMA,
        ],
        compiler_params=pltpu.CompilerParams(use_tc_tiling_on_sc=False),
    )
    def k(table_hbm, idx_hbm, out_hbm, idx_v, rows_v, sem):
        wid = lax.axis_index("s") * 2 + lax.axis_index("c")
        base = wid * rpw
        pltpu.sync_copy(idx_hbm.at[pl.ds(wid * nch, nch)], idx_v)

        def chunk(j, _):
            pltpu.async_copy(table_hbm.at[idx_v.at[j]], rows_v, sem).wait()
            pltpu.sync_copy(rows_v, out_hbm.at[pl.ds(base + j * CH, CH)])
            return 0

        lax.fori_loop(0, nch, chunk, 0, unroll=False)

    return k(table, idx2)


_EPS = 1e-5

def _mxu_dot(a, b):
    return jnp.dot(a.astype(jnp.bfloat16), b.astype(jnp.bfloat16),
                   preferred_element_type=jnp.float32)


def _bn_affine(mean_in, gram_in, wt, bvec, g, be, m_count):
    mu_in = mean_in / m_count
    mu = jnp.dot(mu_in, wt, preferred_element_type=jnp.float32) + bvec
    t = jnp.dot(gram_in / m_count, wt, preferred_element_type=jnp.float32)
    ex2 = (jnp.sum(wt * t, axis=0, keepdims=True)
           + 2.0 * bvec * (mu - bvec) + bvec * bvec)
    var = ex2 - mu * mu
    scale = g / jnp.sqrt(var + _EPS)
    shift = be - mu * scale
    return scale, shift


def _k1_body(g_ref, c_ref, f_ref, sum_ref, gram_ref):
    R = g_ref.shape[0]
    rows = R // _NSAMPLE
    c = c_ref[...]
    cmask = jnp.where(
        lax.broadcasted_iota(jnp.int32, c.shape, 1) < 3, c, 0.0)
    f = (g_ref[...].reshape(rows, _NSAMPLE, 16) - cmask[:, None, :])
    f2 = f.reshape(R, 16)
    f_ref[...] = f2

    @pl.when(pl.program_id(0) == 0)
    def _():
        sum_ref[...] = jnp.zeros_like(sum_ref)
        gram_ref[...] = jnp.zeros_like(gram_ref)

    sum_ref[...] += jnp.sum(f2, axis=0, keepdims=True)
    gram_ref[...] += lax.dot_general(
        f2, f2, (((0,), (0,)), ((), ())),
        preferred_element_type=jnp.float32)


def _stats_body(nlayer, f_ref, w1_ref, b1_ref, g1_ref, be1_ref,
                w2_ref, b2_ref, g2_ref, be2_ref,
                sumf_ref, gramf_ref, s1_ref, g1m_ref,
                sy_ref, gy_ref, m_count=None):
    y = f_ref[...]
    s, t = _bn_affine(sumf_ref[...], gramf_ref[...], w1_ref[...],
                      b1_ref[...], g1_ref[...], be1_ref[...], m_count)
    y = jnp.maximum(
        _mxu_dot(y, w1_ref[...]) * s + t,
        0.0)
    if nlayer == 2:
        s, t = _bn_affine(s1_ref[...], g1m_ref[...], w2_ref[...],
                          b2_ref[...], g2_ref[...], be2_ref[...], m_count)
        y = jnp.maximum(
            _mxu_dot(y, w2_ref[...]) * s
            + t, 0.0)

    @pl.when(pl.program_id(0) == 0)
    def _():
        sy_ref[...] = jnp.zeros_like(sy_ref)
        gy_ref[...] = jnp.zeros_like(gy_ref)

    sy_ref[...] += jnp.sum(y, axis=0, keepdims=True)
    gy_ref[...] += lax.dot_general(
        y, y, (((0,), (0,)), ((), ())), preferred_element_type=jnp.float32)


def _final_body(f_ref, w1_ref, b1_ref, g1_ref, be1_ref,
                w2_ref, b2_ref, g2_ref, be2_ref,
                w3_ref, b3_ref, g3_ref, be3_ref,
                sumf_ref, gramf_ref, s1_ref, g1m_ref, s2_ref, g2m_ref,
                out_ref, m_count=None):
    y = f_ref[...]
    s, t = _bn_affine(sumf_ref[...], gramf_ref[...], w1_ref[...],
                      b1_ref[...], g1_ref[...], be1_ref[...], m_count)
    y = jnp.maximum(
        _mxu_dot(y, w1_ref[...]) * s + t,
        0.0)
    s, t = _bn_affine(s1_ref[...], g1m_ref[...], w2_ref[...],
                      b2_ref[...], g2_ref[...], be2_ref[...], m_count)
    y = jnp.maximum(
        _mxu_dot(y, w2_ref[...]) * s + t,
        0.0)
    s, t = _bn_affine(s2_ref[...], g2m_ref[...], w3_ref[...],
                      b3_ref[...], g3_ref[...], be3_ref[...], m_count)
    y = jnp.maximum(
        _mxu_dot(y, w3_ref[...]) * s + t,
        0.0)
    R = y.shape[0]
    out_ref[...] = jnp.max(
        y.reshape(R // _NSAMPLE, _NSAMPLE, y.shape[1]), axis=1)


def _run_mlp(G, C, w1t, b1, g1, be1, w2t, b2, g2, be2, w3t, b3, g3, be3):
    M = G.shape[0]
    R = 8192
    nblk = M // R
    rows = R // _NSAMPLE
    fM = float(M)

    f, sumf, gramf = pl.pallas_call(
        _k1_body,
        grid=(nblk,),
        in_specs=[
            pl.BlockSpec((R, 16), lambda i: (i, 0)),
            pl.BlockSpec((rows, 16), lambda i: (i, 0)),
        ],
        out_specs=[
            pl.BlockSpec((R, 16), lambda i: (i, 0)),
            pl.BlockSpec((1, 16), lambda i: (0, 0)),
            pl.BlockSpec((16, 16), lambda i: (0, 0)),
        ],
        out_shape=[
            jax.ShapeDtypeStruct((M, 16), jnp.float32),
            jax.ShapeDtypeStruct((1, 16), jnp.float32),
            jax.ShapeDtypeStruct((16, 16), jnp.float32),
        ],
        interpret=_INTERPRET,
    )(G, C)

    def full(shape):
        return pl.BlockSpec(shape, lambda i: tuple(0 for _ in shape))

    w_specs1 = [full((16, 32)), full((1, 32)), full((1, 32)), full((1, 32))]
    w_specs2 = [full((32, 32)), full((1, 32)), full((1, 32)), full((1, 32))]
    w_specs3 = [full((32, 64)), full((1, 64)), full((1, 64)), full((1, 64))]
    st16 = [full((1, 16)), full((16, 16))]
    st32 = [full((1, 32)), full((32, 32))]
    f_spec = pl.BlockSpec((R, 16), lambda i: (i, 0))

    zz32 = jnp.zeros((1, 32), jnp.float32)
    zg32 = jnp.zeros((32, 32), jnp.float32)

    s1, g1m = pl.pallas_call(
        functools.partial(_stats_body, 1, m_count=fM),
        grid=(nblk,),
        in_specs=[f_spec] + w_specs1 + w_specs2 + st16 + st32,
        out_specs=[full((1, 32)), full((32, 32))],
        out_shape=[
            jax.ShapeDtypeStruct((1, 32), jnp.float32),
            jax.ShapeDtypeStruct((32, 32), jnp.float32),
        ],
        interpret=_INTERPRET,
    )(f, w1t, b1, g1, be1, w2t, b2, g2, be2, sumf, gramf, zz32, zg32)

    s2, g2m = pl.pallas_call(
        functools.partial(_stats_body, 2, m_count=fM),
        grid=(nblk,),
        in_specs=[f_spec] + w_specs1 + w_specs2 + st16 + st32,
        out_specs=[full((1, 32)), full((32, 32))],
        out_shape=[
            jax.ShapeDtypeStruct((1, 32), jnp.float32),
            jax.ShapeDtypeStruct((32, 32), jnp.float32),
        ],
        interpret=_INTERPRET,
    )(f, w1t, b1, g1, be1, w2t, b2, g2, be2, sumf, gramf, s1, g1m)

    out = pl.pallas_call(
        functools.partial(_final_body, m_count=fM),
        grid=(nblk,),
        in_specs=([f_spec] + w_specs1 + w_specs2 + w_specs3
                  + st16 + st32 + st32),
        out_specs=pl.BlockSpec((rows, 64), lambda i: (i, 0)),
        out_shape=jax.ShapeDtypeStruct((M // _NSAMPLE, 64), jnp.float32),
        interpret=_INTERPRET,
    )(f, w1t, b1, g1, be1, w2t, b2, g2, be2, w3t, b3, g3, be3,
      sumf, gramf, s1, g1m, s2, g2m)
    return out


def kernel(xyz, points, W1, b1, g1, be1, W2, b2, g2, be2, W3, b3, g3, be3):
    B, N, _ = xyz.shape
    S = _NPOINT

    f0 = jax.random.randint(jax.random.key(1), (B,), 0, N).astype(jnp.int32)
    xyzT = jnp.transpose(xyz, (2, 0, 1))
    xyzT4 = xyzT.reshape(3, B, 8, N // 8)
    fps_flat = _run_fps(xyzT4, f0.reshape(1, B))
    fps_flat = jnp.transpose(fps_flat, (1, 0)).reshape(B * S)

    table = jnp.concatenate(
        [xyz, points, jnp.zeros((B, N, 7), jnp.float32)], axis=-1
    ).reshape(B * N, 16)

    c_rows = _gather_rows(table, fps_flat)
    new_xyz = c_rows[:, :3].reshape(B, S, 3)

    ball_idx = _run_select(
        c_rows.reshape(B, S, 16), jnp.transpose(xyz, (0, 2, 1)))
    g_rows = _gather_rows(table, ball_idx.reshape(B * S * _NSAMPLE))

    pad = jnp.zeros((7, 32), jnp.float32)
    w1t = jnp.concatenate([W1.T, pad], axis=0)
    out = _run_mlp(
        g_rows, c_rows, w1t,
        b1.reshape(1, 32), g1.reshape(1, 32), be1.reshape(1, 32),
        W2.T, b2.reshape(1, 32), g2.reshape(1, 32), be2.reshape(1, 32),
        W3.T, b3.reshape(1, 64), g3.reshape(1, 64), be3.reshape(1, 64))
    new_points = out.reshape(B, S, 64)
    return new_xyz, new_points


# speedup vs baseline: 20.8183x; 1.0163x over previous
import functools

import jax
import jax.numpy as jnp
from jax import lax
from jax.experimental import pallas as pl
from jax.experimental.pallas import tpu as pltpu
from jax.experimental.pallas import tpu_sc as plsc

_NPOINT = 1024
_NSAMPLE = 32
_RADIUS = 0.2
_INTERPRET = False

_BIGKEY = 0x7FFFFFFF
_IDXMASK = 8191
_KEYCLEAR = -8192


def _fps_body(xyzT_ref, f0_ref, nxyz_ref):
    _, B, SUB, NL = xyzT_ref.shape
    N = SUB * NL
    x0 = xyzT_ref[0]
    x1 = xyzT_ref[1]
    x2 = xyzT_ref[2]
    s_iota = lax.broadcasted_iota(jnp.int32, (B, SUB, NL), 1)
    l_iota = lax.broadcasted_iota(jnp.int32, (B, SUB, NL), 2)
    nflat = s_iota * NL + l_iota

    def _redmax(v):
        return jnp.max(jnp.max(v, axis=2, keepdims=True), axis=1, keepdims=True)

    def _redmin(v):
        return jnp.min(jnp.min(v, axis=2, keepdims=True), axis=1, keepdims=True)

    def step(i, carry):
        f, dist = carry
        onehot = (nflat == f)[None]
        cc = jnp.max(jnp.max(jnp.where(onehot, xyzT_ref[...], -jnp.inf),
                             axis=3, keepdims=True), axis=2, keepdims=True)
        nxyz_ref[:, pl.ds(i, 1), :] = jnp.concatenate(
            [cc[0], cc[1], cc[2]], axis=2)
        d = ((x0 - cc[0]) ** 2 + (x1 - cc[1]) ** 2 + (x2 - cc[2]) ** 2)
        dist = jnp.minimum(dist, d)
        m = _redmax(dist)
        cand = jnp.where(dist == m, nflat, N)
        return _redmin(cand), dist

    f0 = f0_ref[0, :].reshape(B, 1, 1)
    dist0 = jnp.full((B, SUB, NL), 1e10, jnp.float32)
    lax.fori_loop(0, _NPOINT, step, (f0, dist0), unroll=2)


def _run_fps(xyzT, f0):
    B = xyzT.shape[1]
    return pl.pallas_call(
        _fps_body,
        out_shape=jax.ShapeDtypeStruct((B, _NPOINT, 3), jnp.float32),
        interpret=_INTERPRET,
    )(xyzT, f0)


def _select_body(q_ref, xyzT_ref, out_ref, key_ref):
    SB = q_ref.shape[1]
    N = xyzT_ref.shape[2]
    b = pl.program_id(0)
    x0 = xyzT_ref[0, 0:1, :]
    x1 = xyzT_ref[0, 1:2, :]
    x2 = xyzT_ref[0, 2:3, :]
    q0 = q_ref[0, :, 0:1]
    q1 = q_ref[0, :, 1:2]
    q2 = q_ref[0, :, 2:3]
    x2sum = x0 * x0 + x1 * x1 + x2 * x2
    q2sum = q0 * q0 + q1 * q1 + q2 * q2

    def rne16(v):
        bv = lax.bitcast_convert_type(v, jnp.int32)
        r = bv + 0x7FFF + jnp.bitwise_and(lax.shift_right_arithmetic(bv, 16), 1)
        return lax.bitcast_convert_type(jnp.bitwise_and(r, -65536), jnp.float32)

    acc = (rne16(q0) * rne16(x0) + rne16(q1) * rne16(x1)
           + rne16(q2) * rne16(x2))
    d2 = (q2sum + x2sum) - 2.0 * acc
    d = jnp.sqrt(jnp.maximum(d2, 0.0))
    n_iota = lax.broadcasted_iota(jnp.int32, (SB, N), 1)
    bits = lax.bitcast_convert_type(d, jnp.int32)
    key_all = jnp.bitwise_or(jnp.bitwise_and(bits, _KEYCLEAR), n_iota)
    nearest = jnp.bitwise_and(
        jnp.min(key_all, axis=1, keepdims=True), _IDXMASK)
    key = jnp.where(d <= jnp.float32(_RADIUS), key_all, _BIGKEY)
    key_ref[...] = key
    col = lax.broadcasted_iota(jnp.int32, (SB, _NSAMPLE), 1)
    out0 = jnp.zeros((SB, _NSAMPLE), jnp.int32)
    kprev0 = jnp.full((SB, 1), -1, jnp.int32)

    def step(k, carry):
        out, kprev = carry
        kk = key_ref[...]
        kmin = jnp.min(jnp.where(kk > kprev, kk, _BIGKEY),
                       axis=1, keepdims=True)
        sel = jnp.where(kmin != _BIGKEY,
                        jnp.bitwise_and(kmin, _IDXMASK), nearest)
        out = jnp.where(col == k, sel, out)
        return out, kmin

    out, _ = lax.fori_loop(0, _NSAMPLE, step, (out0, kprev0), unroll=False)
    out_ref[0] = out + b * N


def _run_select(q_rows, xyzT2, SB=256):
    B, S, _ = q_rows.shape
    N = xyzT2.shape[2]
    return pl.pallas_call(
        _select_body,
        grid=(B, S // SB),
        in_specs=[
            pl.BlockSpec((1, SB, 3), lambda b, s: (b, s, 0)),
            pl.BlockSpec((1, 3, N), lambda b, s: (b, 0, 0)),
        ],
        out_specs=pl.BlockSpec((1, SB, _NSAMPLE), lambda b, s: (b, s, 0)),
        out_shape=jax.ShapeDtypeStruct((B, S, _NSAMPLE), jnp.int32),
        scratch_shapes=[pltpu.VMEM((SB, N), jnp.int32)],
        interpret=_INTERPRET,
    )(q_rows, xyzT2)


def _gather_rows(table, idx):
    M = idx.shape[0]
    D = table.shape[1]
    NW = 32
    rpw = M // NW
    CH = min(128, rpw)
    nch = rpw // CH
    mesh = plsc.VectorSubcoreMesh(core_axis_name="c", subcore_axis_name="s")
    idx2 = idx.reshape(M // CH, CH)

    @functools.partial(
        pl.kernel,
        out_type=jax.ShapeDtypeStruct((M, D), jnp.float32),
        mesh=mesh,
        scratch_types=[
            pltpu.VMEM((nch, CH), jnp.int32),
            pltpu.VMEM((CH, D), jnp.float32),
            pltpu.SemaphoreType.DMA,
        ],
        compiler_params=pltpu.CompilerParams(use_tc_tiling_on_sc=False),
    )
    def k(table_hbm, idx_hbm, out_hbm, idx_v, rows_v, sem):
        wid = lax.axis_index("s") * 2 + lax.axis_index("c")
        base = wid * rpw
        pltpu.sync_copy(idx_hbm.at[pl.ds(wid * nch, nch)], idx_v)

        def chunk(j, _):
            pltpu.async_copy(table_hbm.at[idx_v.at[j]], rows_v, sem).wait()
            pltpu.sync_copy(rows_v, out_hbm.at[pl.ds(base + j * CH, CH)])
            return 0

        lax.fori_loop(0, nch, chunk, 0, unroll=False)

    return k(table, idx2)


_EPS = 1e-5

def _mxu_dot(a, b):
    return jnp.dot(a.astype(jnp.bfloat16), b.astype(jnp.bfloat16),
                   preferred_element_type=jnp.float32)


def _bn_affine(mean_in, gram_in, wt, bvec, g, be, m_count):
    mu_in = mean_in / m_count
    mu = jnp.dot(mu_in, wt, preferred_element_type=jnp.float32) + bvec
    t = jnp.dot(gram_in / m_count, wt, preferred_element_type=jnp.float32)
    ex2 = (jnp.sum(wt * t, axis=0, keepdims=True)
           + 2.0 * bvec * (mu - bvec) + bvec * bvec)
    var = ex2 - mu * mu
    scale = g / jnp.sqrt(var + _EPS)
    shift = be - mu * scale
    return scale, shift


def _mlp_body(g_ref, c_ref, w1_ref, b1_ref, g1_ref, be1_ref,
              w2_ref, b2_ref, g2_ref, be2_ref,
              w3_ref, b3_ref, g3_ref, be3_ref, out_ref,
              sumf, gramf, s1, g1m, s2, g2m, m_count=None, R=None):
    p = pl.program_id(0)
    i = pl.program_id(1)
    rows = R // _NSAMPLE

    def feat():
        c = c_ref[...]
        cpad = jnp.concatenate(
            [c, jnp.zeros((rows, 13), jnp.float32)], axis=1)
        f = (g_ref[...].reshape(rows, _NSAMPLE, 16) - cpad[:, None, :])
        return f.reshape(R, 16)

    @pl.when(p == 0)
    def _phase0():
        f2 = feat()

        @pl.when(i == 0)
        def _():
            sumf[...] = jnp.zeros_like(sumf)
            gramf[...] = jnp.zeros_like(gramf)

        sumf[...] += jnp.sum(f2, axis=0, keepdims=True)
        gramf[...] += lax.dot_general(
            f2, f2, (((0,), (0,)), ((), ())),
            preferred_element_type=jnp.float32)

    def layer(y, w_ref, b_ref, gg_ref, be_ref, mean_in, gram_in):
        s, t = _bn_affine(mean_in, gram_in, w_ref[...], b_ref[...],
                          gg_ref[...], be_ref[...], m_count)
        return jnp.maximum(_mxu_dot(y, w_ref[...]) * s + t, 0.0)

    @pl.when(p == 1)
    def _phase1():
        y1 = layer(feat(), w1_ref, b1_ref, g1_ref,
                   be1_ref, sumf[...], gramf[...])

        @pl.when(i == 0)
        def _():
            s1[...] = jnp.zeros_like(s1)
            g1m[...] = jnp.zeros_like(g1m)

        s1[...] += jnp.sum(y1, axis=0, keepdims=True)
        g1m[...] += lax.dot_general(
            y1, y1, (((0,), (0,)), ((), ())),
            preferred_element_type=jnp.float32)

    @pl.when(p == 2)
    def _phase2():
        y1 = layer(feat(), w1_ref, b1_ref, g1_ref,
                   be1_ref, sumf[...], gramf[...])
        y2 = layer(y1, w2_ref, b2_ref, g2_ref, be2_ref, s1[...], g1m[...])

        @pl.when(i == 0)
        def _():
            s2[...] = jnp.zeros_like(s2)
            g2m[...] = jnp.zeros_like(g2m)

        s2[...] += jnp.sum(y2, axis=0, keepdims=True)
        g2m[...] += lax.dot_general(
            y2, y2, (((0,), (0,)), ((), ())),
            preferred_element_type=jnp.float32)

    @pl.when(p == 3)
    def _phase3():
        y1 = layer(feat(), w1_ref, b1_ref, g1_ref,
                   be1_ref, sumf[...], gramf[...])
        y2 = layer(y1, w2_ref, b2_ref, g2_ref, be2_ref, s1[...], g1m[...])
        y3 = layer(y2, w3_ref, b3_ref, g3_ref, be3_ref, s2[...], g2m[...])
        out_ref[...] = jnp.max(
            y3.reshape(rows, _NSAMPLE, y3.shape[1]), axis=1)


def _run_mlp(G, C, w1t, b1, g1, be1, w2t, b2, g2, be2, w3t, b3, g3, be3):
    M = G.shape[0]
    R = 8192
    nblk = M // R
    rows = R // _NSAMPLE

    def full(shape):
        return pl.BlockSpec(shape, lambda p, i: tuple(0 for _ in shape))

    return pl.pallas_call(
        functools.partial(_mlp_body, m_count=float(M), R=R),
        grid=(4, nblk),
        in_specs=[
            pl.BlockSpec((R, 16), lambda p, i: (i, 0)),
            pl.BlockSpec((rows, 3), lambda p, i: (i, 0)),
            full((16, 32)), full((1, 32)), full((1, 32)), full((1, 32)),
            full((32, 32)), full((1, 32)), full((1, 32)), full((1, 32)),
            full((32, 64)), full((1, 64)), full((1, 64)), full((1, 64)),
        ],
        out_specs=pl.BlockSpec((rows, 64), lambda p, i: (i, 0)),
        out_shape=jax.ShapeDtypeStruct((M // _NSAMPLE, 64), jnp.float32),
        scratch_shapes=[
            pltpu.VMEM((1, 16), jnp.float32),
            pltpu.VMEM((16, 16), jnp.float32),
            pltpu.VMEM((1, 32), jnp.float32),
            pltpu.VMEM((32, 32), jnp.float32),
            pltpu.VMEM((1, 32), jnp.float32),
            pltpu.VMEM((32, 32), jnp.float32),
        ],
        interpret=_INTERPRET,
    )(G, C, w1t, b1, g1, be1, w2t, b2, g2, be2, w3t, b3, g3, be3)


def kernel(xyz, points, W1, b1, g1, be1, W2, b2, g2, be2, W3, b3, g3, be3):
    B, N, _ = xyz.shape
    S = _NPOINT

    f0 = jax.random.randint(jax.random.key(1), (B,), 0, N).astype(jnp.int32)
    xyzT = jnp.transpose(xyz, (2, 0, 1))
    new_xyz = _run_fps(xyzT.reshape(3, B, 8, N // 8), f0.reshape(1, B))

    ball_idx = _run_select(new_xyz, jnp.transpose(xyz, (0, 2, 1)))

    table = jnp.concatenate(
        [xyz, points, jnp.zeros((B, N, 7), jnp.float32)], axis=-1
    ).reshape(B * N, 16)
    g_rows = _gather_rows(table, ball_idx.reshape(B * S * _NSAMPLE))

    pad = jnp.zeros((7, 32), jnp.float32)
    w1t = jnp.concatenate([W1.T, pad], axis=0)
    out = _run_mlp(
        g_rows, new_xyz.reshape(B * S, 3), w1t,
        b1.reshape(1, 32), g1.reshape(1, 32), be1.reshape(1, 32),
        W2.T, b2.reshape(1, 32), g2.reshape(1, 32), be2.reshape(1, 32),
        W3.T, b3.reshape(1, 64), g3.reshape(1, 64), be3.reshape(1, 64))
    new_points = out.reshape(B, S, 64)
    return new_xyz, new_points
